# baseline (speedup 1.0000x reference)
import jax
import jax.numpy as jnp
from jax import lax
from jax.experimental import pallas as pl
from jax.experimental.pallas import tpu as pltpu
from jax.experimental.pallas import tpu_sc as plsc

N = 10000
E = 320000
H = 128
HF = 64
AD = 8
MH = 128
EB = 4000
NB = 2000
NTILE = 16
E2 = E // 2
EPT = E2 // NTILE
W = 80
NCHUNK = EPT // W
RB = 40
NRB = N // RB
F32 = jnp.float32


def _dot(a, b):
    return jnp.dot(a, b, preferred_element_type=F32)


_W128 = pl.BlockSpec((H, H), lambda i: (0, 0))
_NBLK = pl.BlockSpec((NB, H), lambda i: (i, 0))
_NBLK2 = pl.BlockSpec((2, NB, H), lambda i: (0, i, 0))
_NODE_OUT_SHAPE = [
    jax.ShapeDtypeStruct((N, H), F32),
    jax.ShapeDtypeStruct((2, N, H), F32),
    jax.ShapeDtypeStruct((2, N, H), F32),
    jax.ShapeDtypeStruct((N, H), F32),
]
_NODE_OUT_SPECS = [_NBLK, _NBLK2, _NBLK2, _NBLK]


def _write_tables(hh, a_ref, b_ref, v_ref, u_ref, td_ref, ts_ref, hu_ref):
    ha = _dot(hh, a_ref[...])
    hb = _dot(hh, b_ref[...])
    vh = _dot(hh, v_ref[...])
    hu_ref[...] = _dot(hh, u_ref[...])
    td_ref[0] = ha
    td_ref[1] = jnp.concatenate([ha[:, HF:], ha[:, :HF]], axis=1)
    ts_ref[0, :, :HF] = hb[:, :HF]
    ts_ref[0, :, HF:] = vh[:, :HF]
    ts_ref[1, :, :HF] = hb[:, HF:]
    ts_ref[1, :, HF:] = vh[:, HF:]


def _prep0_body(h0_ref, wn_ref, a_ref, b_ref, v_ref, u_ref,
                h_ref, td_ref, ts_ref, hu_ref):
    hh = _dot(h0_ref[...], wn_ref[...])
    h_ref[...] = hh
    _write_tables(hh, a_ref, b_ref, v_ref, u_ref, td_ref, ts_ref, hu_ref)


_prep0 = pl.pallas_call(
    _prep0_body,
    grid=(N // NB,),
    in_specs=[_NBLK, _W128, _W128, _W128, _W128, _W128],
    out_specs=_NODE_OUT_SPECS,
    out_shape=_NODE_OUT_SHAPE,
)


def _nd_ratio(nda_ref, ndb_ref):
    nd = nda_ref[...] + ndb_ref[...]
    num = jnp.concatenate([nd[0, :, :HF], nd[1, :, :HF]], axis=1)
    den = jnp.concatenate([nd[0, :, HF:], nd[1, :, HF:]], axis=1) + 1e-6
    return num / den


def _update_body(h_ref, hu_ref, nda_ref, ndb_ref, a_ref, b_ref, v_ref, u_ref,
                 h1_ref, td_ref, ts_ref, hu1_ref):
    hh = h_ref[...] + jnp.maximum(hu_ref[...] + _nd_ratio(nda_ref, ndb_ref),
                                  0.0)
    h1_ref[...] = hh
    _write_tables(hh, a_ref, b_ref, v_ref, u_ref, td_ref, ts_ref, hu1_ref)


_update1 = pl.pallas_call(
    _update_body,
    grid=(N // NB,),
    in_specs=[_NBLK, _NBLK, _NBLK2, _NBLK2, _W128, _W128, _W128, _W128],
    out_specs=_NODE_OUT_SPECS,
    out_shape=_NODE_OUT_SHAPE,
)


def _edge_init_body(e0_ref, we_ref, c0_ref, ec0_ref):
    ep = _dot(e0_ref[...], we_ref[...])
    ec = _dot(ep, c0_ref[...])
    ec0_ref[0] = ec[:, :HF]
    ec0_ref[1] = ec[:, HF:]


_edge_init = pl.pallas_call(
    _edge_init_body,
    grid=(E2 // EB,),
    in_specs=[
        pl.BlockSpec((EB, 16), lambda i: (i, 0)),
        pl.BlockSpec((16, H), lambda i: (0, 0)),
        pl.BlockSpec((H, H), lambda i: (0, 0)),
    ],
    out_specs=[pl.BlockSpec((2, EB, HF), lambda i: (0, i, 0))],
    out_shape=[jax.ShapeDtypeStruct((2, E2, HF), F32)],
)


def _edgemm1_body(e0_ref, we_ref, g_ref, c1_ref, ec1_ref):
    ep = _dot(e0_ref[...], we_ref[...])
    g = jnp.concatenate([g_ref[0], g_ref[1]], axis=1)
    e1 = ep + jnp.maximum(g, 0.0)
    ec = _dot(e1, c1_ref[...])
    ec1_ref[0] = ec[:, :HF]
    ec1_ref[1] = ec[:, HF:]


_edgemm1 = pl.pallas_call(
    _edgemm1_body,
    grid=(E2 // EB,),
    in_specs=[
        pl.BlockSpec((EB, 16), lambda i: (i, 0)),
        pl.BlockSpec((16, H), lambda i: (0, 0)),
        pl.BlockSpec((2, EB, HF), lambda i: (0, i, 0)),
        pl.BlockSpec((H, H), lambda i: (0, 0)),
    ],
    out_specs=[pl.BlockSpec((2, EB, HF), lambda i: (0, i, 0))],
    out_shape=[jax.ShapeDtypeStruct((2, E2, HF), F32)],
)


def _head_body(h_ref, hu_ref, nda_ref, ndb_ref, act_ref, w1_ref, b1_ref,
               w2_ref, b2_ref, out_ref, sacc_ref):
    i = pl.program_id(0)

    @pl.when(i == 0)
    def _():
        sacc_ref[...] = jnp.zeros_like(sacc_ref)

    hh = h_ref[...] + jnp.maximum(hu_ref[...] + _nd_ratio(nda_ref, ndb_ref),
                                  0.0)
    r = jnp.maximum(
        _dot(hh, w1_ref[0:H, :]) + _dot(act_ref[...], w1_ref[H:H + AD, :])
        + b1_ref[...], 0.0)
    zz = _dot(r, w2_ref[...]) + b2_ref[...]
    sacc_ref[...] += jnp.sum(zz, axis=0, keepdims=True)

    @pl.when(i == N // NB - 1)
    def _():
        out_ref[...] = sacc_ref[...] * (1.0 / N)


_head = pl.pallas_call(
    _head_body,
    grid=(N // NB,),
    in_specs=[
        _NBLK, _NBLK, _NBLK2, _NBLK2,
        pl.BlockSpec((NB, AD), lambda i: (i, 0)),
        pl.BlockSpec((H + AD, H), lambda i: (0, 0)),
        pl.BlockSpec((1, H), lambda i: (0, 0)),
        pl.BlockSpec((H, 1), lambda i: (0, 0)),
        pl.BlockSpec((1, 1), lambda i: (0, 0)),
    ],
    out_specs=pl.BlockSpec((1, 1), lambda i: (0, 0)),
    out_shape=jax.ShapeDtypeStruct((1, 1), F32),
    scratch_shapes=[pltpu.VMEM((1, 1), F32)],
)


MPAIR = NCHUNK // 2


def _make_sc_edge(write_g):
    mesh = plsc.VectorSubcoreMesh(core_axis_name="c", subcore_axis_name="s")
    outs = [jax.ShapeDtypeStruct((2 * N, H), F32)]
    if write_g:
        outs = [jax.ShapeDtypeStruct((2 * E2, HF), F32)] + outs
    scratch = [
        pltpu.VMEM((W,), jnp.int32),
        pltpu.VMEM((W,), jnp.int32),
        pltpu.VMEM((W,), jnp.int32),
        pltpu.VMEM((W,), jnp.int32),
        pltpu.VMEM((W, HF), F32),
        pltpu.VMEM((W, H), F32),
        pltpu.VMEM((W, H), F32),
        pltpu.VMEM((W, H), F32),
        pltpu.VMEM_SHARED((N, H), F32),
        pltpu.SemaphoreType.DMA,
        pltpu.SemaphoreType.DMA,
        pltpu.SemaphoreType.DMA,
        pltpu.SemaphoreType.DMA,
        pltpu.SemaphoreType.DMA,
        pltpu.SemaphoreType.DMA,
    ]

    def body(dst_hbm, src_hbm, ec_hbm, td_hbm, ts_hbm, *rest):
        if write_g:
            g_hbm, nd_hbm = rest[0], rest[1]
            rest = rest[2:]
        else:
            nd_hbm = rest[0]
            rest = rest[1:]
        (dstvA, dstvB, gsxA, gsxB, ecv, tdv, tsv, payv,
         acc, sidx, sec, std, sts, ssc, sgw) = rest
        c = lax.axis_index("c")
        s = lax.axis_index("s")

        zv = jnp.zeros((16,), F32)

        def zero_row(r, cc):
            for j in range(H // 16):
                payv[r, pl.ds(j * 16, 16)] = zv
            return cc

        lax.fori_loop(0, RB, zero_row, 0)
        for bi in range((NRB + NTILE - 1) // NTILE):
            b = s + bi * NTILE

            @pl.when(b < NRB)
            def _():
                pltpu.sync_copy(payv.at[pl.ds(0, RB)],
                                acc.at[pl.ds(pl.multiple_of(b * RB, 8), RB)])

        plsc.subcore_barrier()
        tile_base = s * EPT

        def cb(k):
            return pl.multiple_of(c * E2 + tile_base + k * W, 8)

        def idx_issue(k, dstv, gsx):
            base = pl.multiple_of(tile_base + k * W, 8)
            pltpu.async_copy(dst_hbm.at[pl.ds(base, W)], dstv, sidx)
            pltpu.async_copy(src_hbm.at[pl.ds(base, W)], gsx, sidx)

        def idx_wait(dstv, gsx):
            pltpu.make_async_copy(dst_hbm.at[pl.ds(0, W)], dstv, sidx).wait()
            pltpu.make_async_copy(src_hbm.at[pl.ds(0, W)], gsx, sidx).wait()

        def ec_issue(k):
            pltpu.async_copy(ec_hbm.at[pl.ds(cb(k), W)], ecv, sec)

        def ec_wait():
            pltpu.make_async_copy(ec_hbm.at[pl.ds(0, W)], ecv, sec).wait()

        def gather_issue(gdx, gsx):
            pltpu.async_copy(td_hbm.at[c].at[gdx], tdv, std)
            pltpu.async_copy(ts_hbm.at[c].at[gsx], tsv, sts)

        def gather_wait(gdx, gsx):
            pltpu.make_async_copy(td_hbm.at[c].at[gdx], tdv, std).wait()
            pltpu.make_async_copy(ts_hbm.at[c].at[gsx], tsv, sts).wait()

        def compute():
            def edge_body(i2, cc):
                i0 = 2 * i2
                for ii in (i0, i0 + 1):
                    for j in range(HF // 16):
                        sl = pl.ds(j * 16, 16)
                        sh = pl.ds(HF + j * 16, 16)
                        g = (tdv[ii, sl] + tsv[ii, sl]) + ecv[ii, sl]
                        sg = 1.0 / (1.0 + jnp.exp(-g))
                        payv[ii, sl] = sg * tsv[ii, sh]
                        payv[ii, sh] = sg
                        if write_g:
                            ecv[ii, sl] = g
                return cc

            lax.fori_loop(0, W // 2, edge_body, 0)

        def scatter_issue(dstv):
            pltpu.async_copy(payv, acc.at[dstv], ssc, add=True)

        def scatter_wait(dstv):
            pltpu.make_async_copy(payv, acc.at[dstv], ssc).wait()

        def g_issue(k):
            pltpu.async_copy(ecv, g_hbm.at[pl.ds(cb(k), W)], sgw)

        def g_wait():
            pltpu.make_async_copy(ecv, g_hbm.at[pl.ds(0, W)], sgw).wait()

        def chunk(k, cur, nxt, first, last):
            dstv, gsx = cur

            @pl.when(jnp.logical_not(first))
            def _():
                scatter_wait(nxt[0])
                if write_g:
                    g_wait()

            ec_issue(k)

            @pl.when(jnp.logical_not(last))
            def _():
                idx_issue(k + 1, *nxt)

            gather_wait(dstv, gsx)
            ec_wait()
            compute()
            scatter_issue(dstv)
            if write_g:
                g_issue(k)

            @pl.when(jnp.logical_not(last))
            def _():
                idx_wait(*nxt)
                gather_issue(nxt[0], nxt[1])

        tb = pl.multiple_of(tile_base, 8)
        pltpu.sync_copy(dst_hbm.at[pl.ds(tb, W)], dstvA)
        pltpu.sync_copy(src_hbm.at[pl.ds(tb, W)], gsxA)
        gather_issue(dstvA, gsxA)

        setA = (dstvA, gsxA)
        setB = (dstvB, gsxB)

        def pair_loop(m, carry):
            k0 = 2 * m
            chunk(k0, setA, setB, m == 0, False)
            chunk(k0 + 1, setB, setA, False, False)
            return carry

        lax.fori_loop(0, MPAIR, pair_loop, 0)

        chunk(NCHUNK - 1, setA, setB, False, True)
        scatter_wait(dstvA)
        if write_g:
            g_wait()

        plsc.subcore_barrier()
        for bi in range((NRB + NTILE - 1) // NTILE):
            b = s + bi * NTILE

            @pl.when(b < NRB)
            def _():
                row = pl.multiple_of(b * RB, 8)
                pltpu.sync_copy(acc.at[pl.ds(row, RB)], payv.at[pl.ds(0, RB)])
                pltpu.sync_copy(
                    payv.at[pl.ds(0, RB)],
                    nd_hbm.at[pl.ds(pl.multiple_of(c * N + b * RB, 8), RB)])

    return pl.kernel(body, mesh=mesh, out_type=outs, scratch_types=scratch)


_sc_edge_g = _make_sc_edge(True)
_sc_edge_nog = _make_sc_edge(False)


def kernel(h, e, edge_index, action, Wn, We, A, B, C, U, V, W1, b1, W2, b2):
    src = edge_index[0]
    dst = edge_index[1]
    dstA, dstB = dst[:E2], dst[E2:]
    srcA, srcB = src[:E2], src[E2:]
    eA, eB = e[:E2], e[E2:]
    b1r = b1.reshape(1, MH)
    b2r = b2.reshape(1, 1)

    h0p, td0, ts0, hu0 = _prep0(h, Wn, A[0], B[0], V[0], U[0])
    (ec0A,) = _edge_init(eA, We, C[0])
    g0A, nd0A = _sc_edge_g(dstA, srcA, ec0A.reshape(2 * E2, HF), td0, ts0)
    (ec0B,) = _edge_init(eB, We, C[0])
    g0B, nd0B = _sc_edge_g(dstB, srcB, ec0B.reshape(2 * E2, HF), td0, ts0)
    (ec1A,) = _edgemm1(eA, We, g0A.reshape(2, E2, HF), C[1])
    (ec1B,) = _edgemm1(eB, We, g0B.reshape(2, E2, HF), C[1])
    h1, td1, ts1, hu1 = _update1(h0p, hu0, nd0A.reshape(2, N, H),
                                 nd0B.reshape(2, N, H), A[1], B[1], V[1], U[1])
    (nd1A,) = _sc_edge_nog(dstA, srcA, ec1A.reshape(2 * E2, HF), td1, ts1)
    (nd1B,) = _sc_edge_nog(dstB, srcB, ec1B.reshape(2 * E2, HF), td1, ts1)
    return _head(h1, hu1, nd1A.reshape(2, N, H), nd1B.reshape(2, N, H),
                 action, W1, b1r, W2, b2r)


# speedup vs baseline: 1.0108x; 1.0108x over previous
import jax
import jax.numpy as jnp
from jax import lax
from jax.experimental import pallas as pl
from jax.experimental.pallas import tpu as pltpu
from jax.experimental.pallas import tpu_sc as plsc

N = 10000
E = 320000
H = 128
HF = 64
AD = 8
MH = 128
EB = 4000
NB = 2000
NTILE = 16
E2 = E // 2
EPT = E2 // NTILE
W = 80
NCHUNK = EPT // W
RB = 40
NRB = N // RB
F32 = jnp.float32


def _dot(a, b):
    return jnp.dot(a, b, preferred_element_type=F32)


_W128 = pl.BlockSpec((H, H), lambda i: (0, 0))
_NBLK = pl.BlockSpec((NB, H), lambda i: (i, 0))
_NBLK2 = pl.BlockSpec((2, NB, H), lambda i: (0, i, 0))
_NODE_OUT_SHAPE = [
    jax.ShapeDtypeStruct((N, H), F32),
    jax.ShapeDtypeStruct((2, N, H), F32),
    jax.ShapeDtypeStruct((2, N, H), F32),
    jax.ShapeDtypeStruct((N, H), F32),
]
_NODE_OUT_SPECS = [_NBLK, _NBLK2, _NBLK2, _NBLK]


def _write_tables(hh, a_ref, b_ref, v_ref, u_ref, td_ref, ts_ref, hu_ref):
    ha = _dot(hh, a_ref[...])
    hb = _dot(hh, b_ref[...])
    vh = _dot(hh, v_ref[...])
    hu_ref[...] = _dot(hh, u_ref[...])
    td_ref[0] = ha
    td_ref[1] = jnp.concatenate([ha[:, HF:], ha[:, :HF]], axis=1)
    ts_ref[0, :, :HF] = hb[:, :HF]
    ts_ref[0, :, HF:] = vh[:, :HF]
    ts_ref[1, :, :HF] = hb[:, HF:]
    ts_ref[1, :, HF:] = vh[:, HF:]


def _prep0_body(h0_ref, wn_ref, a_ref, b_ref, v_ref, u_ref,
                h_ref, td_ref, ts_ref, hu_ref):
    hh = _dot(h0_ref[...], wn_ref[...])
    h_ref[...] = hh
    _write_tables(hh, a_ref, b_ref, v_ref, u_ref, td_ref, ts_ref, hu_ref)


_prep0 = pl.pallas_call(
    _prep0_body,
    grid=(N // NB,),
    in_specs=[_NBLK, _W128, _W128, _W128, _W128, _W128],
    out_specs=_NODE_OUT_SPECS,
    out_shape=_NODE_OUT_SHAPE,
)


def _nd_ratio(nda_ref, ndb_ref):
    nd = nda_ref[...] + ndb_ref[...]
    num = jnp.concatenate([nd[0, :, :HF], nd[1, :, :HF]], axis=1)
    den = jnp.concatenate([nd[0, :, HF:], nd[1, :, HF:]], axis=1) + 1e-6
    return num / den


def _update_body(h_ref, hu_ref, nda_ref, ndb_ref, a_ref, b_ref, v_ref, u_ref,
                 h1_ref, td_ref, ts_ref, hu1_ref):
    hh = h_ref[...] + jnp.maximum(hu_ref[...] + _nd_ratio(nda_ref, ndb_ref),
                                  0.0)
    h1_ref[...] = hh
    _write_tables(hh, a_ref, b_ref, v_ref, u_ref, td_ref, ts_ref, hu1_ref)


_update1 = pl.pallas_call(
    _update_body,
    grid=(N // NB,),
    in_specs=[_NBLK, _NBLK, _NBLK2, _NBLK2, _W128, _W128, _W128, _W128],
    out_specs=_NODE_OUT_SPECS,
    out_shape=_NODE_OUT_SHAPE,
)


def _edge_init_body(e0_ref, we_ref, c0_ref, ec0_ref):
    ep = _dot(e0_ref[...], we_ref[...])
    ec = _dot(ep, c0_ref[...])
    ec0_ref[0] = ec[:, :HF]
    ec0_ref[1] = ec[:, HF:]


_edge_init = pl.pallas_call(
    _edge_init_body,
    grid=(E2 // EB,),
    in_specs=[
        pl.BlockSpec((EB, 16), lambda i: (i, 0)),
        pl.BlockSpec((16, H), lambda i: (0, 0)),
        pl.BlockSpec((H, H), lambda i: (0, 0)),
    ],
    out_specs=[pl.BlockSpec((2, EB, HF), lambda i: (0, i, 0))],
    out_shape=[jax.ShapeDtypeStruct((2, E2, HF), F32)],
)


def _edgemm1_body(e0_ref, we_ref, g_ref, c1_ref, ec1_ref):
    ep = _dot(e0_ref[...], we_ref[...])
    g = jnp.concatenate([g_ref[0], g_ref[1]], axis=1)
    e1 = ep + jnp.maximum(g, 0.0)
    ec = _dot(e1, c1_ref[...])
    ec1_ref[0] = ec[:, :HF]
    ec1_ref[1] = ec[:, HF:]


_edgemm1 = pl.pallas_call(
    _edgemm1_body,
    grid=(E2 // EB,),
    in_specs=[
        pl.BlockSpec((EB, 16), lambda i: (i, 0)),
        pl.BlockSpec((16, H), lambda i: (0, 0)),
        pl.BlockSpec((2, EB, HF), lambda i: (0, i, 0)),
        pl.BlockSpec((H, H), lambda i: (0, 0)),
    ],
    out_specs=[pl.BlockSpec((2, EB, HF), lambda i: (0, i, 0))],
    out_shape=[jax.ShapeDtypeStruct((2, E2, HF), F32)],
)


def _head_body(h_ref, hu_ref, nda_ref, ndb_ref, act_ref, w1_ref, b1_ref,
               w2_ref, b2_ref, out_ref, sacc_ref):
    i = pl.program_id(0)

    @pl.when(i == 0)
    def _():
        sacc_ref[...] = jnp.zeros_like(sacc_ref)

    hh = h_ref[...] + jnp.maximum(hu_ref[...] + _nd_ratio(nda_ref, ndb_ref),
                                  0.0)
    r = jnp.maximum(
        _dot(hh, w1_ref[0:H, :]) + _dot(act_ref[...], w1_ref[H:H + AD, :])
        + b1_ref[...], 0.0)
    zz = _dot(r, w2_ref[...]) + b2_ref[...]
    sacc_ref[...] += jnp.sum(zz, axis=0, keepdims=True)

    @pl.when(i == N // NB - 1)
    def _():
        out_ref[...] = sacc_ref[...] * (1.0 / N)


_head = pl.pallas_call(
    _head_body,
    grid=(N // NB,),
    in_specs=[
        _NBLK, _NBLK, _NBLK2, _NBLK2,
        pl.BlockSpec((NB, AD), lambda i: (i, 0)),
        pl.BlockSpec((H + AD, H), lambda i: (0, 0)),
        pl.BlockSpec((1, H), lambda i: (0, 0)),
        pl.BlockSpec((H, 1), lambda i: (0, 0)),
        pl.BlockSpec((1, 1), lambda i: (0, 0)),
    ],
    out_specs=pl.BlockSpec((1, 1), lambda i: (0, 0)),
    out_shape=jax.ShapeDtypeStruct((1, 1), F32),
    scratch_shapes=[pltpu.VMEM((1, 1), F32)],
)


MPAIR = NCHUNK // 2


def _make_sc_edge(write_g):
    mesh = plsc.VectorSubcoreMesh(core_axis_name="c", subcore_axis_name="s")
    outs = [jax.ShapeDtypeStruct((2 * N, H), F32)]
    if write_g:
        outs = [jax.ShapeDtypeStruct((2 * E2, HF), F32)] + outs
    scratch = [
        pltpu.VMEM((W,), jnp.int32),
        pltpu.VMEM((W,), jnp.int32),
        pltpu.VMEM((W,), jnp.int32),
        pltpu.VMEM((W,), jnp.int32),
        pltpu.VMEM((W, HF), F32),
        pltpu.VMEM((W, H), F32),
        pltpu.VMEM((W, H), F32),
        pltpu.VMEM((W, H), F32),
        pltpu.VMEM_SHARED((N, H), F32),
        pltpu.SemaphoreType.DMA,
        pltpu.SemaphoreType.DMA,
        pltpu.SemaphoreType.DMA,
        pltpu.SemaphoreType.DMA,
        pltpu.SemaphoreType.DMA,
        pltpu.SemaphoreType.DMA,
    ]

    def body(dst_hbm, src_hbm, ec_hbm, td_hbm, ts_hbm, *rest):
        if write_g:
            g_hbm, nd_hbm = rest[0], rest[1]
            rest = rest[2:]
        else:
            nd_hbm = rest[0]
            rest = rest[1:]
        (dstvA, dstvB, gsxA, gsxB, ecv, tdv, tsv, payv,
         acc, sidx, sec, std, sts, ssc, sgw) = rest
        c = lax.axis_index("c")
        s = lax.axis_index("s")

        zv = jnp.zeros((16,), F32)

        def zero_row(r, cc):
            for j in range(H // 16):
                payv[r, pl.ds(j * 16, 16)] = zv
            return cc

        lax.fori_loop(0, RB, zero_row, 0)
        NBI = (NRB + NTILE - 1) // NTILE
        for bi in range(NBI):
            b = s + bi * NTILE

            @pl.when(b < NRB)
            def _():
                pltpu.async_copy(payv.at[pl.ds(0, RB)],
                                 acc.at[pl.ds(pl.multiple_of(b * RB, 8), RB)],
                                 ssc)

        for bi in range(NBI):
            b = s + bi * NTILE

            @pl.when(b < NRB)
            def _():
                pltpu.make_async_copy(
                    payv.at[pl.ds(0, RB)],
                    acc.at[pl.ds(pl.multiple_of(b * RB, 8), RB)], ssc).wait()

        plsc.subcore_barrier()
        tile_base = s * EPT

        def cb(k):
            return pl.multiple_of(c * E2 + tile_base + k * W, 8)

        def idx_issue(k, dstv, gsx):
            base = pl.multiple_of(tile_base + k * W, 8)
            pltpu.async_copy(dst_hbm.at[pl.ds(base, W)], dstv, sidx)
            pltpu.async_copy(src_hbm.at[pl.ds(base, W)], gsx, sidx)

        def idx_wait(dstv, gsx):
            pltpu.make_async_copy(dst_hbm.at[pl.ds(0, W)], dstv, sidx).wait()
            pltpu.make_async_copy(src_hbm.at[pl.ds(0, W)], gsx, sidx).wait()

        def ec_issue(k):
            pltpu.async_copy(ec_hbm.at[pl.ds(cb(k), W)], ecv, sec)

        def ec_wait():
            pltpu.make_async_copy(ec_hbm.at[pl.ds(0, W)], ecv, sec).wait()

        def gather_issue(gdx, gsx):
            pltpu.async_copy(td_hbm.at[c].at[gdx], tdv, std)
            pltpu.async_copy(ts_hbm.at[c].at[gsx], tsv, sts)

        def gather_wait(gdx, gsx):
            pltpu.make_async_copy(td_hbm.at[c].at[gdx], tdv, std).wait()
            pltpu.make_async_copy(ts_hbm.at[c].at[gsx], tsv, sts).wait()

        def compute():
            def edge_body(i2, cc):
                i0 = 2 * i2
                for ii in (i0, i0 + 1):
                    for j in range(HF // 16):
                        sl = pl.ds(j * 16, 16)
                        sh = pl.ds(HF + j * 16, 16)
                        g = (tdv[ii, sl] + tsv[ii, sl]) + ecv[ii, sl]
                        sg = 1.0 / (1.0 + jnp.exp(-g))
                        payv[ii, sl] = sg * tsv[ii, sh]
                        payv[ii, sh] = sg
                        if write_g:
                            ecv[ii, sl] = g
                return cc

            lax.fori_loop(0, W // 2, edge_body, 0)

        def scatter_issue(dstv):
            pltpu.async_copy(payv, acc.at[dstv], ssc, add=True)

        def scatter_wait(dstv):
            pltpu.make_async_copy(payv, acc.at[dstv], ssc).wait()

        def g_issue(k):
            pltpu.async_copy(ecv, g_hbm.at[pl.ds(cb(k), W)], sgw)

        def g_wait():
            pltpu.make_async_copy(ecv, g_hbm.at[pl.ds(0, W)], sgw).wait()

        def chunk(k, cur, nxt, first, last):
            dstv, gsx = cur

            @pl.when(jnp.logical_not(first))
            def _():
                scatter_wait(nxt[0])
                if write_g:
                    g_wait()

            ec_issue(k)

            @pl.when(jnp.logical_not(last))
            def _():
                idx_issue(k + 1, *nxt)

            gather_wait(dstv, gsx)
            ec_wait()
            compute()
            scatter_issue(dstv)
            if write_g:
                g_issue(k)

            @pl.when(jnp.logical_not(last))
            def _():
                idx_wait(*nxt)
                gather_issue(nxt[0], nxt[1])

        tb = pl.multiple_of(tile_base, 8)
        pltpu.sync_copy(dst_hbm.at[pl.ds(tb, W)], dstvA)
        pltpu.sync_copy(src_hbm.at[pl.ds(tb, W)], gsxA)
        gather_issue(dstvA, gsxA)

        setA = (dstvA, gsxA)
        setB = (dstvB, gsxB)

        def pair_loop(m, carry):
            k0 = 2 * m
            chunk(k0, setA, setB, m == 0, False)
            chunk(k0 + 1, setB, setA, False, False)
            return carry

        lax.fori_loop(0, MPAIR, pair_loop, 0)

        chunk(NCHUNK - 1, setA, setB, False, True)
        scatter_wait(dstvA)
        if write_g:
            g_wait()

        plsc.subcore_barrier()
        NBI2 = (NRB + NTILE - 1) // NTILE

        def fl_src(p):
            return payv.at[pl.ds(p * RB, RB)]

        def fl_dst(b):
            return nd_hbm.at[pl.ds(pl.multiple_of(c * N + b * RB, 8), RB)]

        for bi in range(NBI2):
            b = s + bi * NTILE
            p = bi % 2

            @pl.when(b < NRB)
            def _():
                if bi >= 2:
                    pltpu.make_async_copy(
                        fl_src(p), fl_dst(s + (bi - 2) * NTILE), ssc).wait()
                pltpu.sync_copy(acc.at[pl.ds(pl.multiple_of(b * RB, 8), RB)],
                                fl_src(p))
                pltpu.async_copy(fl_src(p), fl_dst(b), ssc)

        for bi in range(NBI2):
            b = s + bi * NTILE
            b2 = s + (bi + 2) * NTILE

            @pl.when(jnp.logical_and(b < NRB, b2 >= NRB))
            def _():
                pltpu.make_async_copy(fl_src(bi % 2), fl_dst(b), ssc).wait()

    return pl.kernel(body, mesh=mesh, out_type=outs, scratch_types=scratch)


_sc_edge_g = _make_sc_edge(True)
_sc_edge_nog = _make_sc_edge(False)


def kernel(h, e, edge_index, action, Wn, We, A, B, C, U, V, W1, b1, W2, b2):
    src = edge_index[0]
    dst = edge_index[1]
    dstA, dstB = dst[:E2], dst[E2:]
    srcA, srcB = src[:E2], src[E2:]
    eA, eB = e[:E2], e[E2:]
    b1r = b1.reshape(1, MH)
    b2r = b2.reshape(1, 1)

    h0p, td0, ts0, hu0 = _prep0(h, Wn, A[0], B[0], V[0], U[0])
    (ec0A,) = _edge_init(eA, We, C[0])
    g0A, nd0A = _sc_edge_g(dstA, srcA, ec0A.reshape(2 * E2, HF), td0, ts0)
    (ec0B,) = _edge_init(eB, We, C[0])
    g0B, nd0B = _sc_edge_g(dstB, srcB, ec0B.reshape(2 * E2, HF), td0, ts0)
    (ec1A,) = _edgemm1(eA, We, g0A.reshape(2, E2, HF), C[1])
    (ec1B,) = _edgemm1(eB, We, g0B.reshape(2, E2, HF), C[1])
    h1, td1, ts1, hu1 = _update1(h0p, hu0, nd0A.reshape(2, N, H),
                                 nd0B.reshape(2, N, H), A[1], B[1], V[1], U[1])
    (nd1A,) = _sc_edge_nog(dstA, srcA, ec1A.reshape(2 * E2, HF), td1, ts1)
    (nd1B,) = _sc_edge_nog(dstB, srcB, ec1B.reshape(2 * E2, HF), td1, ts1)
    return _head(h1, hu1, nd1A.reshape(2, N, H), nd1B.reshape(2, N, H),
                 action, W1, b1r, W2, b2r)
